# baseline (speedup 1.0000x reference)
import jax
import jax.numpy as jnp
from jax import lax
from jax.experimental import pallas as pl
from jax.experimental.pallas import tpu as pltpu
from jax.experimental.pallas import tpu_sc as plsc

_NC = 2
_NS = 16
_NW = _NC * _NS
_RPC = 16
_COLS = 1024


def _tc_body(q_ref, x_ref, o_ref):
    q0 = q_ref[0]
    qk = q_ref[15]
    inv = 15.0 / (qk - q0)
    step = (qk - q0) * (1.0 / 15.0)
    t = (x_ref[...] - q0) * inv
    t = jnp.clip(jnp.round(t), 0.0, 15.0)
    o_ref[...] = t * step + q0


def _tc_call(x2, qpoints, row0, n_rows):
    br = 512
    return pl.pallas_call(
        _tc_body,
        grid=(n_rows // br,),
        in_specs=[
            pl.BlockSpec(memory_space=pltpu.SMEM),
            pl.BlockSpec((br, _COLS), lambda i: (row0 // 512 + i, 0)),
        ],
        out_specs=pl.BlockSpec((br, _COLS), lambda i: (i, 0)),
        out_shape=jax.ShapeDtypeStruct((n_rows, _COLS), jnp.float32),
    )(qpoints, x2)


def _make_sc_call(n_rows, total_rows):
    rows_per_w = n_rows // _NW
    n_chunks = rows_per_w // _RPC
    assert n_chunks % 2 == 0
    mesh = plsc.VectorSubcoreMesh(core_axis_name="c", subcore_axis_name="s")

    def body(x_hbm, c_hbm, out_hbm, cbuf, ib0, ib1, ob0, ob1,
             si0, si1, so0, so1):
        wid = lax.axis_index("c") * _NS + lax.axis_index("s")
        row0 = wid * rows_per_w
        pltpu.sync_copy(c_hbm, cbuf)
        a = cbuf[pl.ds(0, 16)]
        b = cbuf[pl.ds(16, 16)]
        s = cbuf[pl.ds(32, 16)]
        g = cbuf[pl.ds(48, 16)]
        lo = jnp.full((16,), 0.5, jnp.float32)
        hi = jnp.full((16,), 15.5, jnp.float32)
        ibufs, obufs = (ib0, ib1), (ob0, ob1)
        isems, osems = (si0, si1), (so0, so1)

        def in_slice(ci):
            return x_hbm.at[pl.ds(row0 + ci * _RPC, _RPC), :]

        def out_slice(ci):
            return out_hbm.at[pl.ds(row0 + ci * _RPC, _RPC), :]

        pltpu.async_copy(in_slice(0), ib0, si0)
        pltpu.async_copy(in_slice(1), ib1, si1)

        def group(gi, carry):
            for j in range(2):
                ci = gi * 2 + j
                ib, ob = ibufs[j], obufs[j]
                isem, osem = isems[j], osems[j]
                pltpu.make_async_copy(in_slice(ci), ib, isem).wait()

                @pl.when(ci >= 2)
                def _():
                    pltpu.make_async_copy(ob, out_slice(ci - 2), osem).wait()

                def vec(off):
                    for r in range(_RPC):
                        v = ib[r, pl.ds(off, 16)]
                        t = jnp.minimum(jnp.maximum(v * a + b, lo), hi)
                        k = t.astype(jnp.int32).astype(jnp.float32)
                        ob[r, pl.ds(off, 16)] = k * s + g

                plsc.parallel_loop(0, _COLS, step=16, unroll=2)(vec)

                pltpu.async_copy(ob, out_slice(ci), osem)

                @pl.when(ci + 2 < n_chunks)
                def _():
                    pltpu.async_copy(in_slice(ci + 2), ib, isem)

            return carry

        lax.fori_loop(0, n_chunks // 2, group, 0)

        pltpu.make_async_copy(ob0, out_slice(n_chunks - 2), so0).wait()
        pltpu.make_async_copy(ob1, out_slice(n_chunks - 1), so1).wait()

    del total_rows
    return pl.kernel(
        body,
        out_type=jax.ShapeDtypeStruct((n_rows, _COLS), jnp.float32),
        mesh=mesh,
        scratch_types=[
            pltpu.VMEM((64,), jnp.float32),
            pltpu.VMEM((_RPC, _COLS), jnp.float32),
            pltpu.VMEM((_RPC, _COLS), jnp.float32),
            pltpu.VMEM((_RPC, _COLS), jnp.float32),
            pltpu.VMEM((_RPC, _COLS), jnp.float32),
            pltpu.SemaphoreType.DMA,
            pltpu.SemaphoreType.DMA,
            pltpu.SemaphoreType.DMA,
            pltpu.SemaphoreType.DMA,
        ],
    )


_SC_ROWS = 7168


def kernel(x, qpoints):
    n_rows = x.size // _COLS
    x2 = x.reshape(n_rows, _COLS)

    q0 = jnp.min(qpoints)
    qk = jnp.max(qpoints)
    inv = 15.0 / (qk - q0)
    step = (qk - q0) * (1.0 / 15.0)
    consts = jnp.concatenate([
        jnp.full((16,), inv, jnp.float32),
        jnp.full((16,), 0.5 - q0 * inv, jnp.float32),
        jnp.full((16,), step, jnp.float32),
        jnp.full((16,), q0, jnp.float32),
    ])

    o_sc = _make_sc_call(_SC_ROWS, n_rows)(x2, consts)
    o_tc = _tc_call(x2, qpoints, _SC_ROWS, n_rows - _SC_ROWS)
    out = jnp.concatenate([o_sc, o_tc], axis=0)
    return out.reshape(x.shape)


# speedup vs baseline: 1.1709x; 1.1709x over previous
import jax
import jax.numpy as jnp
from jax import lax
from jax.experimental import pallas as pl
from jax.experimental.pallas import tpu as pltpu
from jax.experimental.pallas import tpu_sc as plsc

_NC = 2
_NS = 16
_NW = _NC * _NS
_RPC = 16
_COLS = 1024


def _make_sc_call(n_rows):
    rows_per_w = n_rows // _NW
    n_chunks = rows_per_w // _RPC
    assert n_chunks % 2 == 0
    mesh = plsc.VectorSubcoreMesh(core_axis_name="c", subcore_axis_name="s")

    def body(x_hbm, q_hbm, out_hbm, qbuf, ib0, ib1, ob0, ob1,
             si0, si1, so0, so1):
        wid = lax.axis_index("c") * _NS + lax.axis_index("s")
        row0 = wid * rows_per_w

        pltpu.sync_copy(q_hbm, qbuf)
        qv = qbuf[...]
        q0 = jnp.full((16,), qv[0], jnp.float32)
        qk = jnp.full((16,), qv[15], jnp.float32)
        inv = 15.0 / (qk - q0)
        a = inv
        b = 0.5 - q0 * inv
        s = (qk - q0) * (1.0 / 15.0)
        g = q0
        lo = jnp.full((16,), 0.5, jnp.float32)
        hi = jnp.full((16,), 15.5, jnp.float32)

        ibufs, obufs = (ib0, ib1), (ob0, ob1)
        isems, osems = (si0, si1), (so0, so1)

        def in_slice(ci):
            return x_hbm.at[pl.ds(row0 + ci * _RPC, _RPC), :]

        def out_slice(ci):
            return out_hbm.at[pl.ds(row0 + ci * _RPC, _RPC), :]

        pltpu.async_copy(in_slice(0), ib0, si0)
        pltpu.async_copy(in_slice(1), ib1, si1)

        def group(gi, carry):
            for j in range(2):
                ci = gi * 2 + j
                ib, ob = ibufs[j], obufs[j]
                isem, osem = isems[j], osems[j]
                pltpu.make_async_copy(in_slice(ci), ib, isem).wait()

                @pl.when(ci >= 2)
                def _():
                    pltpu.make_async_copy(ob, out_slice(ci - 2), osem).wait()

                def vec(off):
                    for r in range(_RPC):
                        v = ib[r, pl.ds(off, 16)]
                        t = jnp.minimum(jnp.maximum(v * a + b, lo), hi)
                        k = t.astype(jnp.int32).astype(jnp.float32)
                        ob[r, pl.ds(off, 16)] = k * s + g

                plsc.parallel_loop(0, _COLS, step=16, unroll=2)(vec)

                pltpu.async_copy(ob, out_slice(ci), osem)

                @pl.when(ci + 2 < n_chunks)
                def _():
                    pltpu.async_copy(in_slice(ci + 2), ib, isem)

            return carry

        lax.fori_loop(0, n_chunks // 2, group, 0)

        pltpu.make_async_copy(ob0, out_slice(n_chunks - 2), so0).wait()
        pltpu.make_async_copy(ob1, out_slice(n_chunks - 1), so1).wait()

    return pl.kernel(
        body,
        out_type=jax.ShapeDtypeStruct((n_rows, _COLS), jnp.float32),
        mesh=mesh,
        scratch_types=[
            pltpu.VMEM((16,), jnp.float32),
            pltpu.VMEM((_RPC, _COLS), jnp.float32),
            pltpu.VMEM((_RPC, _COLS), jnp.float32),
            pltpu.VMEM((_RPC, _COLS), jnp.float32),
            pltpu.VMEM((_RPC, _COLS), jnp.float32),
            pltpu.SemaphoreType.DMA,
            pltpu.SemaphoreType.DMA,
            pltpu.SemaphoreType.DMA,
            pltpu.SemaphoreType.DMA,
        ],
    )


def kernel(x, qpoints):
    n_rows = x.size // _COLS
    x2 = x.reshape(n_rows, _COLS)
    out = _make_sc_call(n_rows)(x2, qpoints)
    return out.reshape(x.shape)
